# baseline (speedup 1.0000x reference)
import functools

import jax
import jax.numpy as jnp
from jax import lax
from jax.experimental import pallas as pl
from jax.experimental.pallas import tpu as pltpu
from jax.experimental.pallas import tpu_sc as plsc

B = 128
N = 8192
M = 512
K = 5
INV_TAU = 16.0
NC = 2
NS = 16
L = 16
NW = NC * NS
RPW = B // NW
CHUNKS = N // L
UNR = 4
NEG = float("-inf")
MININT = -2147483648

_mesh = plsc.VectorSubcoreMesh(core_axis_name="c", subcore_axis_name="s")


def _to_ukey(bits):
    return bits ^ (lax.shift_right_arithmetic(bits, 31) | MININT)


@functools.partial(
    pl.kernel,
    out_type=jax.ShapeDtypeStruct((NW, L), jnp.float32),
    mesh=_mesh,
    compiler_params=pltpu.CompilerParams(needs_layout_passes=False),
    scratch_types=[
        pltpu.VMEM((N,), jnp.float32),
        pltpu.VMEM((N,), jnp.float32),
        pltpu.VMEM((N + L,), jnp.int32),
        pltpu.VMEM((N + L,), jnp.int32),
        pltpu.VMEM((N + L,), jnp.float32),
        pltpu.VMEM((N + L,), jnp.float32),
        pltpu.VMEM((M + 2 * L,), jnp.float32),
        pltpu.VMEM((B,), jnp.int32),
        pltpu.VMEM((256,), jnp.int32),
        pltpu.VMEM((L,), jnp.float32),
        pltpu.SemaphoreType.DMA,
        pltpu.SemaphoreType.DMA,
    ],
)
def _sc_loss_kernel(outputs_hbm, labels_hbm, out_hbm, row_a, row_b,
                    ck_a, ck_b, cv_a, cv_b, vab_v, labels_v, hist_v,
                    stage_v, sem_a, sem_b):
    wid = lax.axis_index("s") * NC + lax.axis_index("c")
    pltpu.sync_copy(labels_hbm, labels_v)
    lane = lax.iota(jnp.int32, L)
    lane_f = lane.astype(jnp.float32)
    zero_f = jnp.zeros((L,), jnp.float32)
    zero_i = jnp.zeros((L,), jnp.int32)
    one_i = jnp.ones((L,), jnp.int32)
    neg_v = jnp.full((L,), NEG, jnp.float32)

    bufs = [(row_a, sem_a), (row_b, sem_b)]
    row0 = wid * RPW
    pend = pltpu.async_copy(outputs_hbm.at[row0], row_a, sem_a)

    pvec = zero_f
    for j in range(RPW):
        r = row0 + j
        row_v = bufs[j % 2][0]
        pend.wait()
        if j + 1 < RPW:
            nbuf, nsem = bufs[(j + 1) % 2]
            pend = pltpu.async_copy(outputs_hbm.at[r + 1], nbuf, nsem)
        lab_vec = plsc.load_gather(labels_v, [jnp.full((L,), r, jnp.int32)])
        t_vec = plsc.load_gather(row_v, [lab_vec])

        for kk in range(16):
            hist_v[pl.ds(kk * L, L)] = zero_i

        def p1_body(i, carry, row_v=row_v):
            for u in range(UNR):
                chunk = row_v[pl.ds(i * (UNR * L) + u * L, L)]
                bits = lax.bitcast_convert_type(chunk, jnp.int32)
                uk = _to_ukey(bits)
                plsc.addupdate_scatter(
                    hist_v, [lax.shift_right_logical(uk, 24)], one_i)
            return carry

        lax.fori_loop(0, CHUNKS // UNR, p1_body, jnp.int32(0))

        t_bits = lax.bitcast_convert_type(t_vec, jnp.int32)
        t_uk = _to_ukey(t_bits)

        def scan_hist(target):
            hs = [hist_v[pl.ds(kk * L, L)] for kk in range(16)]
            ssum = [jnp.sum(h) for h in hs]
            s_ge = jnp.int32(0)
            suffix = [None] * 16
            for kk in reversed(range(16)):
                suffix[kk] = s_ge + ssum[kk]
                s_ge = suffix[kk]
            tgt_vec = jnp.full((L,), target)
            cnt_true = jnp.int32(0)
            for kk in range(16):
                csum = plsc.cumsum(hs[kk])
                tvals = jnp.full((L,), suffix[kk]) - csum + hs[kk]
                cnt_true = cnt_true + plsc.all_reduce_population_count(
                    tvals >= tgt_vec)[0]
            b_star = cnt_true - 1
            b_vec = jnp.full((L,), b_star)
            d_b = plsc.load_gather(hist_v, [b_vec])[0]
            cnt_ge = jnp.int32(0)
            for kk in range(16):
                cnt_ge = cnt_ge + jnp.sum(
                    jnp.where(lane + (kk * L) >= b_vec, hs[kk], zero_i))
            return b_star, target - (cnt_ge - d_b)

        plsc.addupdate_scatter(
            hist_v, [lax.shift_right_logical(t_uk, 24)], -one_i,
            mask=lane == 0)
        target = jnp.int32(M - 1)
        b_star, target = scan_hist(target)
        b_vec = jnp.full((L,), b_star)
        pref = b_star

        def split0_body(i, carry, row_v=row_v, b_vec=b_vec):
            offm, offa = carry
            for u in range(UNR):
                chunk = row_v[pl.ds(i * (UNR * L) + u * L, L)]
                bits = lax.bitcast_convert_type(chunk, jnp.int32)
                uk = _to_ukey(bits)
                byte = lax.shift_right_logical(uk, 24)
                above = byte > b_vec
                plsc.store_compressed(vab_v.at[pl.ds(offa, L)], chunk,
                                      mask=above)
                offa = offa + plsc.all_reduce_population_count(above)[0]
                match = byte == b_vec
                plsc.store_compressed(ck_a.at[pl.ds(offm, L)], uk,
                                      mask=match)
                plsc.store_compressed(cv_a.at[pl.ds(offm, L)], chunk,
                                      mask=match)
                offm = offm + plsc.all_reduce_population_count(match)[0]
            return offm, offa

        nc, offa = lax.fori_loop(0, CHUNKS // UNR, split0_body,
                                 (jnp.int32(0), jnp.int32(0)))

        pairs = [(ck_a, cv_a), (ck_b, cv_b)]
        for d in range(1, 4):
            shift = 24 - 8 * d
            cks, cvs = pairs[(d + 1) % 2]
            ckd, cvd = pairs[d % 2]
            nchunks = lax.shift_right_logical(nc + (L - 1), 4)
            nc_vec = jnp.full((L,), nc)
            for kk in range(16):
                hist_v[pl.ds(kk * L, L)] = zero_i

            def h_body(i, carry, cks=cks, shift=shift, nc_vec=nc_vec):
                uk = cks[pl.ds(i * L, L)]
                valid = (jnp.full((L,), i * L) + lane) < nc_vec
                idx = lax.shift_right_logical(uk, shift) & 0xFF
                plsc.addupdate_scatter(hist_v, [idx], one_i, mask=valid)
                return carry

            lax.fori_loop(0, nchunks, h_body, jnp.int32(0))

            pref_vec = jnp.full((L,), pref)
            tmask = (lane == 0) & (
                lax.shift_right_logical(t_uk, shift + 8) == pref_vec)
            t_idx = lax.shift_right_logical(t_uk, shift) & 0xFF
            plsc.addupdate_scatter(hist_v, [t_idx], -one_i, mask=tmask)

            b_star, target = scan_hist(target)
            b_vec = jnp.full((L,), b_star)
            pref = lax.shift_left(pref, 8) | b_star

            def s_body(i, carry, cks=cks, cvs=cvs, ckd=ckd, cvd=cvd,
                       shift=shift, nc_vec=nc_vec, b_vec=b_vec, d=d):
                offm, offa = carry
                uk = cks[pl.ds(i * L, L)]
                v = cvs[pl.ds(i * L, L)]
                valid = (jnp.full((L,), i * L) + lane) < nc_vec
                byte = lax.shift_right_logical(uk, shift) & 0xFF
                above = valid & (byte > b_vec)
                plsc.store_compressed(vab_v.at[pl.ds(offa, L)], v,
                                      mask=above)
                offa = offa + plsc.all_reduce_population_count(above)[0]
                if d < 3:
                    match = valid & (byte == b_vec)
                    plsc.store_compressed(ckd.at[pl.ds(offm, L)], uk,
                                          mask=match)
                    plsc.store_compressed(cvd.at[pl.ds(offm, L)], v,
                                          mask=match)
                    offm = offm + plsc.all_reduce_population_count(match)[0]
                return offm, offa

            nc, offa = lax.fori_loop(0, nchunks, s_body,
                                     (jnp.int32(0), offa))

        theta_u = jnp.full((L,), pref)
        theta_bits = jnp.where(theta_u < 0, theta_u ^ MININT, ~theta_u)
        theta_vec = lax.bitcast_convert_type(theta_bits, jnp.float32)
        r_f = jnp.full((L,), target).astype(jnp.float32)
        tgt = (t_uk ^ MININT) > (theta_u ^ MININT)

        nab_vec = jnp.full((L,), offa)
        nab_chunks = lax.shift_right_logical(offa + (L - 1), 4)

        def top_body(i, carry):
            s1, s2, s3, s4, s5 = carry
            v = vab_v[pl.ds(i * L, L)]
            valid = (jnp.full((L,), i * L) + lane) < nab_vec
            x = jnp.where(valid, v, NEG)
            n1 = jnp.maximum(s1, x)
            x = jnp.minimum(s1, x)
            n2 = jnp.maximum(s2, x)
            x = jnp.minimum(s2, x)
            n3 = jnp.maximum(s3, x)
            x = jnp.minimum(s3, x)
            n4 = jnp.maximum(s4, x)
            x = jnp.minimum(s4, x)
            n5 = jnp.maximum(s5, x)
            return n1, n2, n3, n4, n5

        svecs = list(lax.fori_loop(0, nab_chunks, top_body, (neg_v,) * 5))
        svecs.append(jnp.where(
            lane < jnp.minimum(jnp.full((L,), target), 5), theta_vec,
            neg_v))
        svecs.append(jnp.where((lane == 0) & jnp.logical_not(tgt), t_vec,
                               neg_v))

        levels = [jnp.full((L,), jnp.max(jnp.maximum(
            jnp.maximum(svecs[0], svecs[5]), svecs[6])))]
        counts = []
        for _p in range(K - 1):
            prev = levels[-1]
            macc, cacc = neg_v, zero_i
            for s in svecs:
                macc = jnp.maximum(macc, jnp.where(s < prev, s, NEG))
                cacc = cacc + jnp.where(s == prev, one_i, zero_i)
            counts.append(jnp.sum(cacc))
            levels.append(jnp.full((L,), jnp.max(macc)))
        cacc = zero_i
        for s in svecs:
            cacc = cacc + jnp.where(s == levels[-1], one_i, zero_i)
        counts.append(jnp.sum(cacc))

        w1 = levels[0]

        def ab_body(i, carry):
            sexp, z0, z1, z2, z3, z4 = carry
            v = vab_v[pl.ds(i * L, L)]
            valid = (jnp.full((L,), i * L) + lane) < nab_vec
            sexp = sexp + jnp.where(valid, jnp.exp(v - w1), 0.0)
            z0 = z0 + jnp.where(
                valid, jnp.exp(-jnp.abs(v - levels[0]) * INV_TAU), 0.0)
            z1 = z1 + jnp.where(
                valid, jnp.exp(-jnp.abs(v - levels[1]) * INV_TAU), 0.0)
            z2 = z2 + jnp.where(
                valid, jnp.exp(-jnp.abs(v - levels[2]) * INV_TAU), 0.0)
            z3 = z3 + jnp.where(
                valid, jnp.exp(-jnp.abs(v - levels[3]) * INV_TAU), 0.0)
            z4 = z4 + jnp.where(
                valid, jnp.exp(-jnp.abs(v - levels[4]) * INV_TAU), 0.0)
            return sexp, z0, z1, z2, z3, z4

        sexp, z0, z1, z2, z3, z4 = lax.fori_loop(
            0, nab_chunks, ab_body, (zero_f,) * 6)
        zsums = [jnp.full((L,), jnp.sum(z)) for z in (z0, z1, z2, z3, z4)]
        sexp_v = jnp.full((L,), jnp.sum(sexp))

        et = jnp.exp(t_vec - w1)
        sum_exp = (sexp_v + jnp.where(tgt, zero_f, et)
                   + r_f * jnp.exp(theta_vec - w1))
        pv = 0.2 * et / sum_exp

        cum = jnp.int32(0)
        for p in range(K):
            a_v = jnp.full((L,), cum)
            cum = cum + counts[p]
            b_v = jnp.full((L,), cum)
            cmask = (lane >= a_v) & (lane < b_v) & (lane < K)
            coeff_lane = 0.2 * jnp.minimum(float(K) - lane_f, 4.0)
            cs_v = jnp.full((L,), jnp.sum(jnp.where(cmask, coeff_lane, 0.0)))
            numer = jnp.exp(-jnp.abs(t_vec - levels[p]) * INV_TAU)
            z_tot = (zsums[p] + jnp.where(tgt, zero_f, numer)
                     + r_f * jnp.exp(-jnp.abs(theta_vec - levels[p])
                                     * INV_TAU))
            pv = pv + cs_v * numer / jnp.maximum(z_tot, 1e-30)

        pvec = jnp.where(lane == j, pv, pvec)

    stage_v[...] = pvec
    pltpu.sync_copy(stage_v, out_hbm.at[wid])


def kernel(outputs, labels):
    p2d = _sc_loss_kernel(outputs, labels)
    p = p2d[:, :RPW].reshape(B)
    return jnp.mean(-jnp.log(p * (1.0 - 2e-07) + 1e-07))


# speedup vs baseline: 1.1826x; 1.1826x over previous
import functools

import jax
import jax.numpy as jnp
from jax import lax
from jax.experimental import pallas as pl
from jax.experimental.pallas import tpu as pltpu
from jax.experimental.pallas import tpu_sc as plsc

B = 128
N = 8192
M = 512
K = 5
INV_TAU = 16.0
NC = 2
NS = 16
L = 16
NW = NC * NS
RPW = B // NW
CHUNKS = N // L
UNR = 4
NEG = float("-inf")
MININT = -2147483648

_mesh = plsc.VectorSubcoreMesh(core_axis_name="c", subcore_axis_name="s")


def _to_ukey(bits):
    return bits ^ (lax.shift_right_arithmetic(bits, 31) | MININT)


@functools.partial(
    pl.kernel,
    out_type=jax.ShapeDtypeStruct((NW, L), jnp.float32),
    mesh=_mesh,
    compiler_params=pltpu.CompilerParams(needs_layout_passes=False),
    scratch_types=[
        pltpu.VMEM((N,), jnp.float32),
        pltpu.VMEM((N,), jnp.float32),
        pltpu.VMEM((N + L,), jnp.int32),
        pltpu.VMEM((N + L,), jnp.int32),
        pltpu.VMEM((N + L,), jnp.float32),
        pltpu.VMEM((N + L,), jnp.float32),
        pltpu.VMEM((M + 2 * L,), jnp.float32),
        pltpu.VMEM((B,), jnp.int32),
        pltpu.VMEM((256,), jnp.int32),
        pltpu.VMEM((L,), jnp.float32),
        pltpu.SemaphoreType.DMA,
        pltpu.SemaphoreType.DMA,
    ],
)
def _sc_loss_kernel(outputs_hbm, labels_hbm, out_hbm, row_a, row_b,
                    ck_a, ck_b, cv_a, cv_b, vab_v, labels_v, hist_v,
                    stage_v, sem_a, sem_b):
    wid = lax.axis_index("s") * NC + lax.axis_index("c")
    pltpu.sync_copy(labels_hbm, labels_v)
    lane = lax.iota(jnp.int32, L)
    lane_f = lane.astype(jnp.float32)
    zero_f = jnp.zeros((L,), jnp.float32)
    zero_i = jnp.zeros((L,), jnp.int32)
    one_i = jnp.ones((L,), jnp.int32)
    neg_v = jnp.full((L,), NEG, jnp.float32)

    bufs = [(row_a, sem_a), (row_b, sem_b)]
    row0 = wid * RPW
    pend = pltpu.async_copy(outputs_hbm.at[row0], row_a, sem_a)

    pvec = zero_f
    for j in range(RPW):
        r = row0 + j
        row_v = bufs[j % 2][0]
        pend.wait()
        if j + 1 < RPW:
            nbuf, nsem = bufs[(j + 1) % 2]
            pend = pltpu.async_copy(outputs_hbm.at[r + 1], nbuf, nsem)
        lab_vec = plsc.load_gather(labels_v, [jnp.full((L,), r, jnp.int32)])
        t_vec = plsc.load_gather(row_v, [lab_vec])

        for kk in range(16):
            hist_v[pl.ds(kk * L, L)] = zero_i

        def p1_body(i, carry, row_v=row_v):
            for u in range(UNR):
                chunk = row_v[pl.ds(i * (UNR * L) + u * L, L)]
                bits = lax.bitcast_convert_type(chunk, jnp.int32)
                uk = _to_ukey(bits)
                plsc.addupdate_scatter(
                    hist_v, [lax.shift_right_logical(uk, 24)], one_i)
            return carry

        lax.fori_loop(0, CHUNKS // UNR, p1_body, jnp.int32(0))

        t_bits = lax.bitcast_convert_type(t_vec, jnp.int32)
        t_uk = _to_ukey(t_bits)

        def scan_hist(target):
            hs = [hist_v[pl.ds(kk * L, L)] for kk in range(16)]
            ssum = [jnp.sum(h) for h in hs]
            s_ge = jnp.int32(0)
            suffix = [None] * 16
            for kk in reversed(range(16)):
                suffix[kk] = s_ge + ssum[kk]
                s_ge = suffix[kk]
            tgt_vec = jnp.full((L,), target)
            cnt_true = jnp.int32(0)
            for kk in range(16):
                csum = plsc.cumsum(hs[kk])
                tvals = jnp.full((L,), suffix[kk]) - csum + hs[kk]
                cnt_true = cnt_true + plsc.all_reduce_population_count(
                    tvals >= tgt_vec)[0]
            b_star = cnt_true - 1
            b_vec = jnp.full((L,), b_star)
            d_b = plsc.load_gather(hist_v, [b_vec])[0]
            cnt_ge = jnp.int32(0)
            for kk in range(16):
                cnt_ge = cnt_ge + jnp.sum(
                    jnp.where(lane + (kk * L) >= b_vec, hs[kk], zero_i))
            return b_star, target - (cnt_ge - d_b)

        plsc.addupdate_scatter(
            hist_v, [lax.shift_right_logical(t_uk, 24)], -one_i,
            mask=lane == 0)
        target = jnp.int32(M - 1)
        b_star, target = scan_hist(target)
        b_vec = jnp.full((L,), b_star)
        pref = b_star

        def split0_body(i, carry, row_v=row_v, b_vec=b_vec):
            offm, offa = carry
            chunks, uks, aboves, matches = [], [], [], []
            offas, offms = [], []
            for u in range(UNR):
                chunk = row_v[pl.ds(i * (UNR * L) + u * L, L)]
                bits = lax.bitcast_convert_type(chunk, jnp.int32)
                uk = _to_ukey(bits)
                byte = lax.shift_right_logical(uk, 24)
                above = byte > b_vec
                match = byte == b_vec
                chunks.append(chunk)
                uks.append(uk)
                aboves.append(above)
                matches.append(match)
                offas.append(offa)
                offms.append(offm)
                offa = offa + plsc.all_reduce_population_count(above)[0]
                offm = offm + plsc.all_reduce_population_count(match)[0]
            for u in range(UNR):
                plsc.store_compressed(vab_v.at[pl.ds(offas[u], L)],
                                      chunks[u], mask=aboves[u])
                plsc.store_compressed(ck_a.at[pl.ds(offms[u], L)], uks[u],
                                      mask=matches[u])
                plsc.store_compressed(cv_a.at[pl.ds(offms[u], L)],
                                      chunks[u], mask=matches[u])
            return offm, offa

        nc, offa = lax.fori_loop(0, CHUNKS // UNR, split0_body,
                                 (jnp.int32(0), jnp.int32(0)))

        pairs = [(ck_a, cv_a), (ck_b, cv_b)]
        for d in range(1, 4):
            shift = 24 - 8 * d
            cks, cvs = pairs[(d + 1) % 2]
            ckd, cvd = pairs[d % 2]
            nchunks = lax.shift_right_logical(nc + (L - 1), 4)
            nc_vec = jnp.full((L,), nc)
            for kk in range(16):
                hist_v[pl.ds(kk * L, L)] = zero_i

            def h_body(i, carry, cks=cks, shift=shift, nc_vec=nc_vec):
                uk = cks[pl.ds(i * L, L)]
                valid = (jnp.full((L,), i * L) + lane) < nc_vec
                idx = lax.shift_right_logical(uk, shift) & 0xFF
                plsc.addupdate_scatter(hist_v, [idx], one_i, mask=valid)
                return carry

            lax.fori_loop(0, nchunks, h_body, jnp.int32(0))

            pref_vec = jnp.full((L,), pref)
            tmask = (lane == 0) & (
                lax.shift_right_logical(t_uk, shift + 8) == pref_vec)
            t_idx = lax.shift_right_logical(t_uk, shift) & 0xFF
            plsc.addupdate_scatter(hist_v, [t_idx], -one_i, mask=tmask)

            b_star, target = scan_hist(target)
            b_vec = jnp.full((L,), b_star)
            pref = lax.shift_left(pref, 8) | b_star

            def s_body(i, carry, cks=cks, cvs=cvs, ckd=ckd, cvd=cvd,
                       shift=shift, nc_vec=nc_vec, b_vec=b_vec, d=d):
                offm, offa = carry
                uk = cks[pl.ds(i * L, L)]
                v = cvs[pl.ds(i * L, L)]
                valid = (jnp.full((L,), i * L) + lane) < nc_vec
                byte = lax.shift_right_logical(uk, shift) & 0xFF
                above = valid & (byte > b_vec)
                plsc.store_compressed(vab_v.at[pl.ds(offa, L)], v,
                                      mask=above)
                offa = offa + plsc.all_reduce_population_count(above)[0]
                if d < 3:
                    match = valid & (byte == b_vec)
                    plsc.store_compressed(ckd.at[pl.ds(offm, L)], uk,
                                          mask=match)
                    plsc.store_compressed(cvd.at[pl.ds(offm, L)], v,
                                          mask=match)
                    offm = offm + plsc.all_reduce_population_count(match)[0]
                return offm, offa

            nc, offa = lax.fori_loop(0, nchunks, s_body,
                                     (jnp.int32(0), offa))

        theta_u = jnp.full((L,), pref)
        theta_bits = jnp.where(theta_u < 0, theta_u ^ MININT, ~theta_u)
        theta_vec = lax.bitcast_convert_type(theta_bits, jnp.float32)
        r_f = jnp.full((L,), target).astype(jnp.float32)
        tgt = (t_uk ^ MININT) > (theta_u ^ MININT)

        nab_vec = jnp.full((L,), offa)
        nab_chunks = lax.shift_right_logical(offa + (L - 1), 4)

        def top_body(i, carry):
            s1, s2, s3, s4, s5 = carry
            v = vab_v[pl.ds(i * L, L)]
            valid = (jnp.full((L,), i * L) + lane) < nab_vec
            x = jnp.where(valid, v, NEG)
            n1 = jnp.maximum(s1, x)
            x = jnp.minimum(s1, x)
            n2 = jnp.maximum(s2, x)
            x = jnp.minimum(s2, x)
            n3 = jnp.maximum(s3, x)
            x = jnp.minimum(s3, x)
            n4 = jnp.maximum(s4, x)
            x = jnp.minimum(s4, x)
            n5 = jnp.maximum(s5, x)
            return n1, n2, n3, n4, n5

        svecs = list(lax.fori_loop(0, nab_chunks, top_body, (neg_v,) * 5))
        svecs.append(jnp.where(
            lane < jnp.minimum(jnp.full((L,), target), 5), theta_vec,
            neg_v))
        svecs.append(jnp.where((lane == 0) & jnp.logical_not(tgt), t_vec,
                               neg_v))

        levels = [jnp.full((L,), jnp.max(jnp.maximum(
            jnp.maximum(svecs[0], svecs[5]), svecs[6])))]
        counts = []
        for _p in range(K - 1):
            prev = levels[-1]
            macc, cacc = neg_v, zero_i
            for s in svecs:
                macc = jnp.maximum(macc, jnp.where(s < prev, s, NEG))
                cacc = cacc + jnp.where(s == prev, one_i, zero_i)
            counts.append(jnp.sum(cacc))
            levels.append(jnp.full((L,), jnp.max(macc)))
        cacc = zero_i
        for s in svecs:
            cacc = cacc + jnp.where(s == levels[-1], one_i, zero_i)
        counts.append(jnp.sum(cacc))

        w1 = levels[0]

        def ab_body(i, carry):
            sexp, z0, z1, z2, z3, z4 = carry
            v = vab_v[pl.ds(i * L, L)]
            valid = (jnp.full((L,), i * L) + lane) < nab_vec
            sexp = sexp + jnp.where(valid, jnp.exp(v - w1), 0.0)
            z0 = z0 + jnp.where(
                valid, jnp.exp(-jnp.abs(v - levels[0]) * INV_TAU), 0.0)
            z1 = z1 + jnp.where(
                valid, jnp.exp(-jnp.abs(v - levels[1]) * INV_TAU), 0.0)
            z2 = z2 + jnp.where(
                valid, jnp.exp(-jnp.abs(v - levels[2]) * INV_TAU), 0.0)
            z3 = z3 + jnp.where(
                valid, jnp.exp(-jnp.abs(v - levels[3]) * INV_TAU), 0.0)
            z4 = z4 + jnp.where(
                valid, jnp.exp(-jnp.abs(v - levels[4]) * INV_TAU), 0.0)
            return sexp, z0, z1, z2, z3, z4

        sexp, z0, z1, z2, z3, z4 = lax.fori_loop(
            0, nab_chunks, ab_body, (zero_f,) * 6)
        zsums = [jnp.full((L,), jnp.sum(z)) for z in (z0, z1, z2, z3, z4)]
        sexp_v = jnp.full((L,), jnp.sum(sexp))

        et = jnp.exp(t_vec - w1)
        sum_exp = (sexp_v + jnp.where(tgt, zero_f, et)
                   + r_f * jnp.exp(theta_vec - w1))
        pv = 0.2 * et / sum_exp

        cum = jnp.int32(0)
        for p in range(K):
            a_v = jnp.full((L,), cum)
            cum = cum + counts[p]
            b_v = jnp.full((L,), cum)
            cmask = (lane >= a_v) & (lane < b_v) & (lane < K)
            coeff_lane = 0.2 * jnp.minimum(float(K) - lane_f, 4.0)
            cs_v = jnp.full((L,), jnp.sum(jnp.where(cmask, coeff_lane, 0.0)))
            numer = jnp.exp(-jnp.abs(t_vec - levels[p]) * INV_TAU)
            z_tot = (zsums[p] + jnp.where(tgt, zero_f, numer)
                     + r_f * jnp.exp(-jnp.abs(theta_vec - levels[p])
                                     * INV_TAU))
            pv = pv + cs_v * numer / jnp.maximum(z_tot, 1e-30)

        pvec = jnp.where(lane == j, pv, pvec)

    stage_v[...] = pvec
    pltpu.sync_copy(stage_v, out_hbm.at[wid])


def kernel(outputs, labels):
    p2d = _sc_loss_kernel(outputs, labels)
    p = p2d[:, :RPW].reshape(B)
    return jnp.mean(-jnp.log(p * (1.0 - 2e-07) + 1e-07))
